# Initial kernel scaffold; baseline (speedup 1.0000x reference)
import functools

import jax
import jax.numpy as jnp
from jax import lax
from jax.experimental import pallas as pl
from jax.experimental.pallas import tpu as pltpu
from jax.experimental.pallas import tpu_sc as plsc

_N = 50000
_E = 800000
_B = 64
_H = 32

_NW = 32
_CHUNK = 128
_K = 4
_CPW = 196
_E_PAD = _NW * _CPW * _CHUNK
_N_PAD = 50048
_EBLK = 1000
_NBLK = 1000

_f32 = jnp.float32


def _gelu(v):
    return 0.5 * v * (1.0 + lax.erf(v / jnp.sqrt(2.0).astype(_f32)))


def _mlp_ln(h, W2, b2, gm, bt):
    h = _gelu(jnp.dot(h, W2, preferred_element_type=_f32) + b2)
    mu = jnp.mean(h, axis=-1, keepdims=True)
    var = jnp.mean((h - mu) ** 2, axis=-1, keepdims=True)
    return (h - mu) / jnp.sqrt(var + 1e-5) * gm + bt


def _xub_body(xb, bb, ub, ob):
    oh = (bb[...] == lax.broadcasted_iota(jnp.int32, (_NBLK, _B), 1)).astype(_f32)
    uu = jnp.dot(oh, ub[...], preferred_element_type=_f32)
    ob[...] = jnp.concatenate([xb[...], uu], axis=1)


def _make_xub(x, batch2d, u):
    return pl.pallas_call(
        _xub_body,
        grid=(_N // _NBLK,),
        in_specs=[
            pl.BlockSpec((_NBLK, _H), lambda i: (i, 0)),
            pl.BlockSpec((_NBLK, 1), lambda i: (i, 0)),
            pl.BlockSpec((_B, _H), lambda i: (0, 0)),
        ],
        out_specs=pl.BlockSpec((_NBLK, 2 * _H), lambda i: (i, 0)),
        out_shape=jax.ShapeDtypeStruct((_N, 2 * _H), _f32),
    )(x, batch2d, u)


def _sc_gather(xub, x, rowg, colg):
    mesh = plsc.VectorSubcoreMesh(core_axis_name="c", subcore_axis_name="s")

    @functools.partial(
        pl.kernel,
        out_type=[
            jax.ShapeDtypeStruct((_E_PAD, 2 * _H), _f32),
            jax.ShapeDtypeStruct((_E_PAD, _H), _f32),
        ],
        mesh=mesh,
        scratch_types=[
            pltpu.VMEM((_K, _CHUNK), jnp.int32),
            pltpu.VMEM((_K, _CHUNK), jnp.int32),
            pltpu.VMEM((_K * _CHUNK, 2 * _H), _f32),
            pltpu.VMEM((_K * _CHUNK, _H), _f32),
            pltpu.SemaphoreType.DMA,
        ],
    )
    def k(xub_h, x_h, rowg_h, colg_h, srcue_o, dst_o, idx_r, idx_c, buf_su, buf_d, sem):
        wid = lax.axis_index("s") * 2 + lax.axis_index("c")

        def step(g, _):
            cbase = wid * _CPW + g * _K
            pltpu.sync_copy(rowg_h.at[pl.ds(cbase, _K)], idx_r)
            pltpu.sync_copy(colg_h.at[pl.ds(cbase, _K)], idx_c)
            cps = []
            for j in range(_K):
                cps.append(pltpu.async_copy(
                    xub_h.at[idx_r.at[j]],
                    buf_su.at[pl.ds(j * _CHUNK, _CHUNK)], sem))
                cps.append(pltpu.async_copy(
                    x_h.at[idx_c.at[j]],
                    buf_d.at[pl.ds(j * _CHUNK, _CHUNK)], sem))
            for c in cps:
                c.wait()
            ebase = cbase * _CHUNK
            pltpu.sync_copy(buf_su, srcue_o.at[pl.ds(ebase, _K * _CHUNK)])
            pltpu.sync_copy(buf_d, dst_o.at[pl.ds(ebase, _K * _CHUNK)])
            return ()

        lax.fori_loop(0, _CPW // _K, step, ())

    return k(xub, x, rowg, colg)


def _edge_body(su, db, eab,
               eW1su, eW1d, eW1e, eb1, eW2, eb2, egm, ebt,
               nW1d, nW1o, nb1, nW2, nb2, ngm, nbt,
               eo, mo):
    d = db[...]
    h1 = (jnp.dot(su[...], eW1su[...], preferred_element_type=_f32)
          + jnp.dot(d, eW1d[...], preferred_element_type=_f32)
          + jnp.dot(eab[...], eW1e[...], preferred_element_type=_f32)
          + eb1[...])
    edge_out = _mlp_ln(_gelu(h1), eW2[...], eb2[...], egm[...], ebt[...])
    eo[...] = edge_out
    m1 = _gelu(jnp.dot(d, nW1d[...], preferred_element_type=_f32)
               + jnp.dot(edge_out, nW1o[...], preferred_element_type=_f32)
               + nb1[...])
    mo[...] = _mlp_ln(m1, nW2[...], nb2[...], ngm[...], nbt[...])


def _make_edge(srcue, dstg, edge_attr, ew, nw):
    wspec = lambda shp: pl.BlockSpec(shp, lambda i: (0, 0))
    return pl.pallas_call(
        _edge_body,
        grid=(_E // _EBLK,),
        in_specs=[
            pl.BlockSpec((_EBLK, 2 * _H), lambda i: (i, 0)),
            pl.BlockSpec((_EBLK, _H), lambda i: (i, 0)),
            pl.BlockSpec((_EBLK, _H), lambda i: (i, 0)),
            wspec((2 * _H, _H)), wspec((_H, _H)), wspec((_H, _H)),
            wspec((1, _H)), wspec((_H, _H)), wspec((1, _H)),
            wspec((1, _H)), wspec((1, _H)),
            wspec((_H, _H)), wspec((_H, _H)), wspec((1, _H)),
            wspec((_H, _H)), wspec((1, _H)), wspec((1, _H)), wspec((1, _H)),
        ],
        out_specs=[
            pl.BlockSpec((_EBLK, _H), lambda i: (i, 0)),
            pl.BlockSpec((_EBLK, _H), lambda i: (i, 0)),
        ],
        out_shape=[
            jax.ShapeDtypeStruct((_E, _H), _f32),
            jax.ShapeDtypeStruct((_E_PAD, _H), _f32),
        ],
    )(srcue, dstg, edge_attr, *ew, *nw)


def _sc_scatter(m, rows_sc, zsum, zcnt, onesb):
    mesh = plsc.VectorSubcoreMesh(core_axis_name="c", subcore_axis_name="s")
    rpt = _N_PAD // 16

    @functools.partial(
        pl.kernel,
        out_type=[
            jax.ShapeDtypeStruct((_N_PAD, _H), _f32),
            jax.ShapeDtypeStruct((_N_PAD, _H), _f32),
            jax.ShapeDtypeStruct((_N_PAD, 8), _f32),
            jax.ShapeDtypeStruct((_N_PAD, 8), _f32),
        ],
        mesh=mesh,
        scratch_types=[
            pltpu.VMEM_SHARED((_N_PAD, _H), _f32),
            pltpu.VMEM_SHARED((_N_PAD, 8), _f32),
            pltpu.VMEM((_K, _CHUNK), jnp.int32),
            pltpu.VMEM((_K * _CHUNK, _H), _f32),
            pltpu.VMEM((_CHUNK, 8), _f32),
        ],
    )
    def k(m_h, rows_h, zsum_h, zcnt_h, ones_h,
          s0_o, s1_o, c0_o, c1_o,
          sh_sum, sh_cnt, idx_v, mval, ones_v):
        cid = lax.axis_index("c")
        sid = lax.axis_index("s")
        wid = sid * 2 + cid
        t0 = sid * rpt
        pltpu.sync_copy(zsum_h.at[pl.ds(t0, rpt)], sh_sum.at[pl.ds(t0, rpt)])
        pltpu.sync_copy(zcnt_h.at[pl.ds(t0, rpt)], sh_cnt.at[pl.ds(t0, rpt)])
        pltpu.sync_copy(ones_h, ones_v)
        plsc.subcore_barrier()

        def step(g, _):
            cbase = wid * _CPW + g * _K
            pltpu.sync_copy(rows_h.at[pl.ds(cbase, _K)], idx_v)
            pltpu.sync_copy(m_h.at[pl.ds(cbase * _CHUNK, _K * _CHUNK)], mval)
            for j in range(_K):
                pltpu.sync_copy(mval.at[pl.ds(j * _CHUNK, _CHUNK)],
                                sh_sum.at[idx_v.at[j]], add=True)
                pltpu.sync_copy(ones_v, sh_cnt.at[idx_v.at[j]], add=True)
            return ()

        lax.fori_loop(0, _CPW // _K, step, ())
        plsc.subcore_barrier()

        @pl.when(cid == 0)
        def _():
            pltpu.sync_copy(sh_sum.at[pl.ds(t0, rpt)], s0_o.at[pl.ds(t0, rpt)])
            pltpu.sync_copy(sh_cnt.at[pl.ds(t0, rpt)], c0_o.at[pl.ds(t0, rpt)])

        @pl.when(cid == 1)
        def _():
            pltpu.sync_copy(sh_sum.at[pl.ds(t0, rpt)], s1_o.at[pl.ds(t0, rpt)])
            pltpu.sync_copy(sh_cnt.at[pl.ds(t0, rpt)], c1_o.at[pl.ds(t0, rpt)])

    return k(m, rows_sc, zsum, zcnt, onesb)


def _node_body(p0, p1, c0, c1, xub, bt_, ub,
               W1a, W1b, b1, W2, b2, gm, bt,
               xo, gso, gco):
    i = pl.program_id(0)
    cnt = jnp.maximum(c0[:, 0:1] + c1[:, 0:1], 1.0)
    agg = (p0[...] + p1[...]) / cnt
    uu = xub[:, _H:2 * _H]
    h1 = _gelu(jnp.dot(agg, W1a[...], preferred_element_type=_f32)
               + jnp.dot(uu, W1b[...], preferred_element_type=_f32)
               + b1[...])
    xn = _mlp_ln(h1, W2[...], b2[...], gm[...], bt[...])
    xo[...] = xn
    ohT = (lax.broadcasted_iota(jnp.int32, (_B, _NBLK), 0) == bt_[...]).astype(_f32)
    gp = jnp.dot(ohT, xn, preferred_element_type=_f32)
    gc = jnp.sum(ohT, axis=1, keepdims=True)

    @pl.when(i == 0)
    def _():
        gso[...] = gp
        gco[...] = gc

    @pl.when(i > 0)
    def _():
        gso[...] += gp
        gco[...] += gc


def _make_node(p0, p1, c0, c1, xub, batchT, u, nw):
    wspec = lambda shp: pl.BlockSpec(shp, lambda i: (0, 0))
    return pl.pallas_call(
        _node_body,
        grid=(_N // _NBLK,),
        in_specs=[
            pl.BlockSpec((_NBLK, _H), lambda i: (i, 0)),
            pl.BlockSpec((_NBLK, _H), lambda i: (i, 0)),
            pl.BlockSpec((_NBLK, 8), lambda i: (i, 0)),
            pl.BlockSpec((_NBLK, 8), lambda i: (i, 0)),
            pl.BlockSpec((_NBLK, 2 * _H), lambda i: (i, 0)),
            pl.BlockSpec((1, _NBLK), lambda i: (0, i)),
            wspec((_B, _H)),
            wspec((_H, _H)), wspec((_H, _H)), wspec((1, _H)),
            wspec((_H, _H)), wspec((1, _H)), wspec((1, _H)), wspec((1, _H)),
        ],
        out_specs=[
            pl.BlockSpec((_NBLK, _H), lambda i: (i, 0)),
            pl.BlockSpec((_B, _H), lambda i: (0, 0)),
            pl.BlockSpec((_B, 1), lambda i: (0, 0)),
        ],
        out_shape=[
            jax.ShapeDtypeStruct((_N, _H), _f32),
            jax.ShapeDtypeStruct((_B, _H), _f32),
            jax.ShapeDtypeStruct((_B, 1), _f32),
        ],
    )(p0, p1, c0, c1, xub, batchT, u, *nw)


def _glob_body(ub, gsb, gcb, W1a, W1b, b1, W2, b2, gm, bt, uo):
    mean = gsb[...] / jnp.maximum(gcb[...], 1.0)
    h1 = _gelu(jnp.dot(ub[...], W1a[...], preferred_element_type=_f32)
               + jnp.dot(mean, W1b[...], preferred_element_type=_f32)
               + b1[...])
    uo[...] = _mlp_ln(h1, W2[...], b2[...], gm[...], bt[...])


def _make_glob(u, gs, gc, gw):
    wspec = lambda shp: pl.BlockSpec(shp, lambda: (0, 0))
    return pl.pallas_call(
        _glob_body,
        grid=(),
        in_specs=[
            wspec((_B, _H)), wspec((_B, _H)), wspec((_B, 1)),
            wspec((_H, _H)), wspec((_H, _H)), wspec((1, _H)),
            wspec((_H, _H)), wspec((1, _H)), wspec((1, _H)), wspec((1, _H)),
        ],
        out_specs=wspec((_B, _H)),
        out_shape=jax.ShapeDtypeStruct((_B, _H), _f32),
    )(u, gs, gc, *gw)


def kernel(x, edge_index, edge_attr, u, batch,
           e_W1, e_b1, e_W2, e_b2, e_gm, e_bt,
           n1_W1, n1_b1, n1_W2, n1_b2, n1_gm, n1_bt,
           n2_W1, n2_b1, n2_W2, n2_b2, n2_gm, n2_bt,
           g_W1, g_b1, g_W2, g_b2, g_gm, g_bt):
    row = edge_index[0]
    col = edge_index[1]
    pad = _E_PAD - _E
    rowg = jnp.concatenate([row, jnp.zeros((pad,), jnp.int32)]).reshape(-1, _CHUNK)
    colg = jnp.concatenate([col, jnp.zeros((pad,), jnp.int32)]).reshape(-1, _CHUNK)
    rows_sc = jnp.concatenate([row, jnp.full((pad,), _N, jnp.int32)]).reshape(-1, _CHUNK)
    batch2d = batch.reshape(_N, 1)
    batchT = batch.reshape(1, _N)
    r2 = lambda v: v.reshape(1, _H)

    xub = _make_xub(x, batch2d, u)
    srcue, dstg = _sc_gather(xub, x, rowg, colg)

    e_W1su = jnp.concatenate([e_W1[0:_H], e_W1[3 * _H:4 * _H]], axis=0)
    ew = (e_W1su, e_W1[_H:2 * _H], e_W1[2 * _H:3 * _H], r2(e_b1),
          e_W2, r2(e_b2), r2(e_gm), r2(e_bt))
    nw = (n1_W1[0:_H], n1_W1[_H:2 * _H], r2(n1_b1),
          n1_W2, r2(n1_b2), r2(n1_gm), r2(n1_bt))
    edge_out, m = _make_edge(srcue, dstg, edge_attr, ew, nw)

    zsum = jnp.zeros((_N_PAD, _H), _f32)
    zcnt = jnp.zeros((_N_PAD, 8), _f32)
    onesb = jnp.ones((_CHUNK, 8), _f32)
    s0, s1, c0, c1 = _sc_scatter(m, rows_sc, zsum, zcnt, onesb)

    n2w = (n2_W1[0:_H], n2_W1[_H:2 * _H], r2(n2_b1),
           n2_W2, r2(n2_b2), r2(n2_gm), r2(n2_bt))
    x_new, gs, gc = _make_node(s0[:_N], s1[:_N], c0[:_N], c1[:_N],
                               xub, batchT, u, n2w)

    gw = (g_W1[0:_H], g_W1[_H:2 * _H], r2(g_b1),
          g_W2, r2(g_b2), r2(g_gm), r2(g_bt))
    u_new = _make_glob(u, gs, gc, gw)

    return (x_new, edge_out, u_new)


# speedup vs baseline: 4.1641x; 4.1641x over previous
import functools

import jax
import jax.numpy as jnp
from jax import lax
from jax.experimental import pallas as pl
from jax.experimental.pallas import tpu as pltpu
from jax.experimental.pallas import tpu_sc as plsc

_N = 50000
_E = 800000
_B = 64
_H = 32

_NW = 32
_CHUNK = 128
_K = 4
_CPW = 196
_E_PAD = _NW * _CPW * _CHUNK
_N_PAD = 50048
_EBLK = 1000
_NBLK = 1000

_f32 = jnp.float32


def _gelu(v):
    return 0.5 * v * (1.0 + lax.erf(v / jnp.sqrt(2.0).astype(_f32)))


def _mlp_ln(h, W2, b2, gm, bt):
    h = _gelu(jnp.dot(h, W2, preferred_element_type=_f32) + b2)
    mu = jnp.mean(h, axis=-1, keepdims=True)
    var = jnp.mean((h - mu) ** 2, axis=-1, keepdims=True)
    return (h - mu) / jnp.sqrt(var + 1e-5) * gm + bt


def _xub_body(xb, bb, ub, ob):
    oh = (bb[...] == lax.broadcasted_iota(jnp.int32, (_NBLK, _B), 1)).astype(_f32)
    uu = jnp.dot(oh, ub[...], preferred_element_type=_f32)
    ob[...] = jnp.concatenate([xb[...], uu], axis=1)


def _make_xub(x, batch2d, u):
    return pl.pallas_call(
        _xub_body,
        grid=(_N // _NBLK,),
        in_specs=[
            pl.BlockSpec((_NBLK, _H), lambda i: (i, 0)),
            pl.BlockSpec((_NBLK, 1), lambda i: (i, 0)),
            pl.BlockSpec((_B, _H), lambda i: (0, 0)),
        ],
        out_specs=pl.BlockSpec((_NBLK, 2 * _H), lambda i: (i, 0)),
        out_shape=jax.ShapeDtypeStruct((_N_PAD, 2 * _H), _f32),
    )(x, batch2d, u)


def _sc_gather(xub, x, rowsc, colg, zcnt, onesb):
    mesh = plsc.VectorSubcoreMesh(core_axis_name="c", subcore_axis_name="s")
    rpt = _N_PAD // 16

    @functools.partial(
        pl.kernel,
        out_type=[
            jax.ShapeDtypeStruct((_E_PAD, 2 * _H), _f32),
            jax.ShapeDtypeStruct((_E_PAD, _H), _f32),
            jax.ShapeDtypeStruct((_N_PAD, 8), _f32),
            jax.ShapeDtypeStruct((_N_PAD, 8), _f32),
        ],
        mesh=mesh,
        scratch_types=[
            pltpu.VMEM_SHARED((_N_PAD, 8), _f32),
            pltpu.VMEM((_K, _CHUNK), jnp.int32),
            pltpu.VMEM((_K, _CHUNK), jnp.int32),
            pltpu.VMEM((_K * _CHUNK, 2 * _H), _f32),
            pltpu.VMEM((_K * _CHUNK, _H), _f32),
            pltpu.VMEM((_CHUNK, 8), _f32),
            pltpu.SemaphoreType.DMA,
        ],
        compiler_params=pltpu.CompilerParams(use_tc_tiling_on_sc=False),
    )
    def k(xub_h, x_h, rowsc_h, colg_h, zcnt_h, ones_h,
          srcue_o, dst_o, c0_o, c1_o,
          sh_cnt, idx_r, idx_c, buf_su, buf_d, ones_v, sem):
        cid = lax.axis_index("c")
        sid = lax.axis_index("s")
        wid = sid * 2 + cid
        t0 = sid * rpt
        pltpu.sync_copy(zcnt_h.at[pl.ds(t0, rpt)], sh_cnt.at[pl.ds(t0, rpt)])
        pltpu.sync_copy(ones_h, ones_v)
        plsc.subcore_barrier()

        def step(g, _):
            cbase = wid * _CPW + g * _K
            pltpu.sync_copy(rowsc_h.at[pl.ds(cbase, _K)], idx_r)
            pltpu.sync_copy(colg_h.at[pl.ds(cbase, _K)], idx_c)
            cps = []
            for j in range(_K):
                cps.append(pltpu.async_copy(
                    xub_h.at[idx_r.at[j]],
                    buf_su.at[pl.ds(j * _CHUNK, _CHUNK)], sem))
                cps.append(pltpu.async_copy(
                    x_h.at[idx_c.at[j]],
                    buf_d.at[pl.ds(j * _CHUNK, _CHUNK)], sem))
            for j in range(_K):
                pltpu.sync_copy(ones_v, sh_cnt.at[idx_r.at[j]], add=True)
            for c in cps:
                c.wait()
            ebase = cbase * _CHUNK
            pltpu.sync_copy(buf_su, srcue_o.at[pl.ds(ebase, _K * _CHUNK)])
            pltpu.sync_copy(buf_d, dst_o.at[pl.ds(ebase, _K * _CHUNK)])
            return ()

        lax.fori_loop(0, _CPW // _K, step, ())
        plsc.subcore_barrier()

        @pl.when(cid == 0)
        def _():
            pltpu.sync_copy(sh_cnt.at[pl.ds(t0, rpt)], c0_o.at[pl.ds(t0, rpt)])

        @pl.when(cid == 1)
        def _():
            pltpu.sync_copy(sh_cnt.at[pl.ds(t0, rpt)], c1_o.at[pl.ds(t0, rpt)])

    return k(xub, x, rowsc, colg, zcnt, onesb)


def _edge_body(su, db, eab,
               eW1su, eW1d, eW1e, eb1, eW2, eb2, egm, ebt,
               nW1d, nW1o, nb1, nW2, nb2, ngm, nbt,
               eo, mo):
    d = db[...]
    h1 = (jnp.dot(su[...], eW1su[...], preferred_element_type=_f32)
          + jnp.dot(d, eW1d[...], preferred_element_type=_f32)
          + jnp.dot(eab[...], eW1e[...], preferred_element_type=_f32)
          + eb1[...])
    edge_out = _mlp_ln(_gelu(h1), eW2[...], eb2[...], egm[...], ebt[...])
    eo[...] = edge_out
    m1 = _gelu(jnp.dot(d, nW1d[...], preferred_element_type=_f32)
               + jnp.dot(edge_out, nW1o[...], preferred_element_type=_f32)
               + nb1[...])
    mo[...] = _mlp_ln(m1, nW2[...], nb2[...], ngm[...], nbt[...])


def _make_edge(srcue, dstg, edge_attr, ew, nw):
    wspec = lambda shp: pl.BlockSpec(shp, lambda i: (0, 0))
    return pl.pallas_call(
        _edge_body,
        grid=(_E // _EBLK,),
        in_specs=[
            pl.BlockSpec((_EBLK, 2 * _H), lambda i: (i, 0)),
            pl.BlockSpec((_EBLK, _H), lambda i: (i, 0)),
            pl.BlockSpec((_EBLK, _H), lambda i: (i, 0)),
            wspec((2 * _H, _H)), wspec((_H, _H)), wspec((_H, _H)),
            wspec((1, _H)), wspec((_H, _H)), wspec((1, _H)),
            wspec((1, _H)), wspec((1, _H)),
            wspec((_H, _H)), wspec((_H, _H)), wspec((1, _H)),
            wspec((_H, _H)), wspec((1, _H)), wspec((1, _H)), wspec((1, _H)),
        ],
        out_specs=[
            pl.BlockSpec((_EBLK, _H), lambda i: (i, 0)),
            pl.BlockSpec((_EBLK, _H), lambda i: (i, 0)),
        ],
        out_shape=[
            jax.ShapeDtypeStruct((_E, _H), _f32),
            jax.ShapeDtypeStruct((_E_PAD, _H), _f32),
        ],
    )(srcue, dstg, edge_attr, *ew, *nw)


def _sc_scatter(m, rows_sc, zsum):
    mesh = plsc.VectorSubcoreMesh(core_axis_name="c", subcore_axis_name="s")
    rpt = _N_PAD // 16

    @functools.partial(
        pl.kernel,
        out_type=[
            jax.ShapeDtypeStruct((_N_PAD, _H), _f32),
            jax.ShapeDtypeStruct((_N_PAD, _H), _f32),
        ],
        mesh=mesh,
        scratch_types=[
            pltpu.VMEM_SHARED((_N_PAD, _H), _f32),
            pltpu.VMEM((_K, _CHUNK), jnp.int32),
            pltpu.VMEM((_K * _CHUNK, _H), _f32),
        ],
        compiler_params=pltpu.CompilerParams(use_tc_tiling_on_sc=False),
    )
    def k(m_h, rows_h, zsum_h,
          s0_o, s1_o,
          sh_sum, idx_v, mval):
        cid = lax.axis_index("c")
        sid = lax.axis_index("s")
        wid = sid * 2 + cid
        t0 = sid * rpt
        pltpu.sync_copy(zsum_h.at[pl.ds(t0, rpt)], sh_sum.at[pl.ds(t0, rpt)])
        plsc.subcore_barrier()

        def step(g, _):
            cbase = wid * _CPW + g * _K
            pltpu.sync_copy(rows_h.at[pl.ds(cbase, _K)], idx_v)
            pltpu.sync_copy(m_h.at[pl.ds(cbase * _CHUNK, _K * _CHUNK)], mval)
            for j in range(_K):
                pltpu.sync_copy(mval.at[pl.ds(j * _CHUNK, _CHUNK)],
                                sh_sum.at[idx_v.at[j]], add=True)
            return ()

        lax.fori_loop(0, _CPW // _K, step, ())
        plsc.subcore_barrier()

        @pl.when(cid == 0)
        def _():
            pltpu.sync_copy(sh_sum.at[pl.ds(t0, rpt)], s0_o.at[pl.ds(t0, rpt)])

        @pl.when(cid == 1)
        def _():
            pltpu.sync_copy(sh_sum.at[pl.ds(t0, rpt)], s1_o.at[pl.ds(t0, rpt)])

    return k(m, rows_sc, zsum)


def _node_body(p0, p1, c0, c1, xub, bt_, ub,
               W1a, W1b, b1, W2, b2, gm, bt,
               xo, gso, gco):
    i = pl.program_id(0)
    cnt = jnp.maximum(c0[:, 0:1] + c1[:, 0:1], 1.0)
    agg = (p0[...] + p1[...]) / cnt
    uu = xub[:, _H:2 * _H]
    h1 = _gelu(jnp.dot(agg, W1a[...], preferred_element_type=_f32)
               + jnp.dot(uu, W1b[...], preferred_element_type=_f32)
               + b1[...])
    xn = _mlp_ln(h1, W2[...], b2[...], gm[...], bt[...])
    xo[...] = xn
    bt_row = bt_[...].reshape(1, _NBLK)
    ohT = (lax.broadcasted_iota(jnp.int32, (_B, _NBLK), 0) == bt_row).astype(_f32)
    gp = jnp.dot(ohT, xn, preferred_element_type=_f32)
    gc = jnp.sum(ohT, axis=1, keepdims=True)

    @pl.when(i == 0)
    def _():
        gso[...] = gp
        gco[...] = gc

    @pl.when(i > 0)
    def _():
        gso[...] += gp
        gco[...] += gc


def _make_node(p0, p1, c0, c1, xub, batchT, u, nw):
    wspec = lambda shp: pl.BlockSpec(shp, lambda i: (0, 0))
    return pl.pallas_call(
        _node_body,
        grid=(_N // _NBLK,),
        in_specs=[
            pl.BlockSpec((_NBLK, _H), lambda i: (i, 0)),
            pl.BlockSpec((_NBLK, _H), lambda i: (i, 0)),
            pl.BlockSpec((_NBLK, 8), lambda i: (i, 0)),
            pl.BlockSpec((_NBLK, 8), lambda i: (i, 0)),
            pl.BlockSpec((_NBLK, 2 * _H), lambda i: (i, 0)),
            pl.BlockSpec((1, 1, _NBLK), lambda i: (i, 0, 0)),
            wspec((_B, _H)),
            wspec((_H, _H)), wspec((_H, _H)), wspec((1, _H)),
            wspec((_H, _H)), wspec((1, _H)), wspec((1, _H)), wspec((1, _H)),
        ],
        out_specs=[
            pl.BlockSpec((_NBLK, _H), lambda i: (i, 0)),
            pl.BlockSpec((_B, _H), lambda i: (0, 0)),
            pl.BlockSpec((_B, 1), lambda i: (0, 0)),
        ],
        out_shape=[
            jax.ShapeDtypeStruct((_N, _H), _f32),
            jax.ShapeDtypeStruct((_B, _H), _f32),
            jax.ShapeDtypeStruct((_B, 1), _f32),
        ],
    )(p0, p1, c0, c1, xub, batchT, u, *nw)


def _glob_body(ub, gsb, gcb, W1a, W1b, b1, W2, b2, gm, bt, uo):
    mean = gsb[...] / jnp.maximum(gcb[...], 1.0)
    h1 = _gelu(jnp.dot(ub[...], W1a[...], preferred_element_type=_f32)
               + jnp.dot(mean, W1b[...], preferred_element_type=_f32)
               + b1[...])
    uo[...] = _mlp_ln(h1, W2[...], b2[...], gm[...], bt[...])


def _make_glob(u, gs, gc, gw):
    return pl.pallas_call(
        _glob_body,
        out_shape=jax.ShapeDtypeStruct((_B, _H), _f32),
    )(u, gs, gc, *gw)


def kernel(x, edge_index, edge_attr, u, batch,
           e_W1, e_b1, e_W2, e_b2, e_gm, e_bt,
           n1_W1, n1_b1, n1_W2, n1_b2, n1_gm, n1_bt,
           n2_W1, n2_b1, n2_W2, n2_b2, n2_gm, n2_bt,
           g_W1, g_b1, g_W2, g_b2, g_gm, g_bt):
    row = edge_index[0]
    col = edge_index[1]
    pad = _E_PAD - _E
    colg = jnp.concatenate([col, jnp.zeros((pad,), jnp.int32)]).reshape(-1, _CHUNK)
    rows_sc = jnp.concatenate([row, jnp.full((pad,), _N, jnp.int32)]).reshape(-1, _CHUNK)
    batch2d = batch.reshape(_N, 1)
    batchT = batch.reshape(_N // _NBLK, 1, _NBLK)
    r2 = lambda v: v.reshape(1, _H)

    zcnt = jnp.zeros((_N_PAD, 8), _f32)
    onesb = jnp.ones((_CHUNK, 8), _f32)
    xub = _make_xub(x, batch2d, u)
    srcue, dstg, c0, c1 = _sc_gather(xub, x, rows_sc, colg, zcnt, onesb)

    e_W1su = jnp.concatenate([e_W1[0:_H], e_W1[3 * _H:4 * _H]], axis=0)
    ew = (e_W1su, e_W1[_H:2 * _H], e_W1[2 * _H:3 * _H], r2(e_b1),
          e_W2, r2(e_b2), r2(e_gm), r2(e_bt))
    nw = (n1_W1[0:_H], n1_W1[_H:2 * _H], r2(n1_b1),
          n1_W2, r2(n1_b2), r2(n1_gm), r2(n1_bt))
    edge_out, m = _make_edge(srcue, dstg, edge_attr, ew, nw)

    zsum = jnp.zeros((_N_PAD, _H), _f32)
    s0, s1 = _sc_scatter(m, rows_sc, zsum)

    n2w = (n2_W1[0:_H], n2_W1[_H:2 * _H], r2(n2_b1),
           n2_W2, r2(n2_b2), r2(n2_gm), r2(n2_bt))
    x_new, gs, gc = _make_node(s0, s1, c0, c1, xub, batchT, u, n2w)

    gw = (g_W1[0:_H], g_W1[_H:2 * _H], r2(g_b1),
          g_W2, r2(g_b2), r2(g_gm), r2(g_bt))
    u_new = _make_glob(u, gs, gc, gw)

    return (x_new, edge_out, u_new)
